# Initial kernel scaffold; baseline (speedup 1.0000x reference)
import functools

import numpy as np
import jax
import jax.numpy as jnp
from jax import lax
from jax.experimental import pallas as pl
from jax.experimental.pallas import tpu as pltpu
from jax.experimental.pallas import tpu_sc as plsc

_NUM_LEVELS = 12
_LOG2_HASH = 14
_HASH = 2 ** _LOG2_HASH
_RES = [int(16 * np.exp((np.log(512) - np.log(16)) / (_NUM_LEVELS - 1)) ** i)
        for i in range(_NUM_LEVELS)]
_N = 1048576
_NC = 2
_NS = 16
_NW = _NC * _NS
_PPW = _N // _NW
_C = 2048
_NCHUNK = _PPW // _C
_G = _C // 16

_P2 = np.int32(np.uint32(2654435761))
_P3 = np.int32(np.uint32(805459861))


def _tec_body(x_hbm, tbl_hbm, out_hbm, xc, idxb, outb, gsem):
    wid = lax.axis_index("s") * _NC + lax.axis_index("c")
    base = wid * _PPW

    def chunk_body(ci, carry):
        pbase = base + ci * _C
        pltpu.sync_copy(x_hbm.at[pl.ds(pbase, _C), :], xc)

        def group_body(g, carry2):
            rows = lax.iota(jnp.int32, 16) + g * 16
            zero = jnp.zeros((16,), jnp.int32)
            xr = plsc.load_gather(xc, [rows, zero])
            yr = plsc.load_gather(xc, [rows, zero + 1])
            zr = plsc.load_gather(xc, [rows, zero + 2])
            posbase = rows * jnp.int32(_NUM_LEVELS)
            for lvl in range(_NUM_LEVELS):
                r = jnp.float32(_RES[lvl])
                ix = (xr * r).astype(jnp.int32)
                iy = (yr * r).astype(jnp.int32)
                iz = (zr * r).astype(jnp.int32)
                h = (ix ^ (iy * _P2) ^ (iz * _P3)) & jnp.int32(_HASH - 1)
                idx = h + jnp.int32(lvl * _HASH)
                plsc.store_scatter(idxb, [posbase + jnp.int32(lvl)], idx)
            return carry2

        lax.fori_loop(0, _G, group_body, 0)
        pltpu.async_copy(tbl_hbm.at[idxb], outb, gsem).wait()
        pltpu.sync_copy(outb, out_hbm.at[pl.ds(pbase * _NUM_LEVELS,
                                               _C * _NUM_LEVELS), :])
        return carry

    lax.fori_loop(0, _NCHUNK, chunk_body, 0)


def kernel(x, tables):
    tbl = tables.reshape(_NUM_LEVELS * _HASH, 2)
    mesh = plsc.VectorSubcoreMesh(core_axis_name="c", subcore_axis_name="s")
    run = pl.kernel(
        _tec_body,
        out_type=jax.ShapeDtypeStruct((_N * _NUM_LEVELS, 2), jnp.float32),
        mesh=mesh,
        scratch_types=[
            pltpu.VMEM((_C, 3), jnp.float32),
            pltpu.VMEM((_C * _NUM_LEVELS,), jnp.int32),
            pltpu.VMEM((_C * _NUM_LEVELS, 2), jnp.float32),
            pltpu.SemaphoreType.DMA,
        ],
    )
    out = run(x, tbl)
    return out.reshape(_N, _NUM_LEVELS * 2)


# speedup vs baseline: 19.0478x; 19.0478x over previous
import numpy as np
import jax
import jax.numpy as jnp
from jax import lax
from jax.experimental import pallas as pl
from jax.experimental.pallas import tpu as pltpu
from jax.experimental.pallas import tpu_sc as plsc

_NUM_LEVELS = 12
_HASH = 2 ** 14
_RES = [int(16 * np.exp((np.log(512) - np.log(16)) / (_NUM_LEVELS - 1)) ** i)
        for i in range(_NUM_LEVELS)]
_N = 1048576
_NC = 2
_NS = 16
_NW = _NC * _NS
_PPW = _N // _NW
_C = 1024
_NCHUNK = _PPW // _C
_G = _C // 16
_NPASS = 3
_LPP = _NUM_LEVELS // _NPASS

_P2 = np.int32(np.uint32(2654435761))
_P3 = np.int32(np.uint32(805459861))


def _tec_body(x_hbm, tbl_hbm, out_hbm, tblv, xc, stage):
    wid = lax.axis_index("s") * np.int32(_NC) + lax.axis_index("c")
    pstart = wid * np.int32(_PPW)

    for p in range(_NPASS):
        pltpu.sync_copy(tbl_hbm.at[pl.ds(p * _LPP * _HASH, _LPP * _HASH)],
                        tblv)

        def chunk_body(ci, carry, p=p):
            pbase = pstart + ci * np.int32(_C)
            pltpu.sync_copy(x_hbm.at[pl.ds(pbase, _C), :], xc)

            def group_body(g, carry2):
                rows = lax.iota(jnp.int32, 16) + g * np.int32(16)
                zero = jnp.zeros((16,), jnp.int32)
                xr = plsc.load_gather(xc, [rows, zero])
                yr = plsc.load_gather(xc, [rows, zero + np.int32(1)])
                zr = plsc.load_gather(xc, [rows, zero + np.int32(2)])
                for j in range(_LPP):
                    r = jnp.float32(_RES[p * _LPP + j])
                    ix = (xr * r).astype(jnp.int32)
                    iy = (yr * r).astype(jnp.int32)
                    iz = (zr * r).astype(jnp.int32)
                    h = ((ix ^ (iy * _P2) ^ (iz * _P3))
                         & jnp.int32(_HASH - 1))
                    packed = plsc.load_gather(tblv, [h + jnp.int32(j * _HASH)])
                    lo = plsc.bitcast(packed << jnp.int32(16), jnp.float32)
                    hi = plsc.bitcast(packed & jnp.int32(-65536), jnp.float32)
                    col = jnp.full((16,), 2 * j, jnp.int32)
                    plsc.store_scatter(stage, [rows, col], lo)
                    plsc.store_scatter(stage, [rows, col + np.int32(1)], hi)
                return carry2

            lax.fori_loop(np.int32(0), np.int32(_G), group_body, np.int32(0))
            pltpu.sync_copy(stage,
                            out_hbm.at[pl.ds(pbase, _C),
                                       pl.ds(p * 2 * _LPP, 2 * _LPP)])
            return carry

        lax.fori_loop(np.int32(0), np.int32(_NCHUNK), chunk_body, np.int32(0))


def _pack_tables(tables):
    tb = jax.lax.bitcast_convert_type(tables.astype(jnp.bfloat16), jnp.uint16)
    packed = (tb[..., 0].astype(jnp.uint32)
              | (tb[..., 1].astype(jnp.uint32) << jnp.uint32(16)))
    return jax.lax.bitcast_convert_type(packed, jnp.int32).reshape(
        _NUM_LEVELS * _HASH)


def kernel(x, tables):
    with jax.enable_x64(False):
        return _run(x, tables)


def _run(x, tables):
    tbl = _pack_tables(tables)
    mesh = plsc.VectorSubcoreMesh(core_axis_name="c", subcore_axis_name="s")
    run = pl.kernel(
        _tec_body,
        out_type=jax.ShapeDtypeStruct((_N, 2 * _NUM_LEVELS), jnp.float32),
        mesh=mesh,
        scratch_types=[
            pltpu.VMEM((_LPP * _HASH,), jnp.int32),
            pltpu.VMEM((_C, 3), jnp.float32),
            pltpu.VMEM((_C, 2 * _LPP), jnp.float32),
        ],
        compiler_params=pltpu.CompilerParams(needs_layout_passes=False,
                                             use_tc_tiling_on_sc=False),
    )
    return run(x, tbl)


# speedup vs baseline: 161.9374x; 8.5016x over previous
import numpy as np
import jax
import jax.numpy as jnp
from jax import lax
from jax.experimental import pallas as pl
from jax.experimental.pallas import tpu as pltpu
from jax.experimental.pallas import tpu_sc as plsc

_NUM_LEVELS = 12
_HASH = 2 ** 14
_RES = [int(16 * np.exp((np.log(512) - np.log(16)) / (_NUM_LEVELS - 1)) ** i)
        for i in range(_NUM_LEVELS)]
_N = 1048576
_NB = _N // 128
_NC = 2
_NS = 16
_NW = _NC * _NS
_PPW = _N // _NW
_C = 2048
_CB = _C // 128
_NCHUNK = _PPW // _C
_G = _C // 16
_NPASS = 3
_LPP = _NUM_LEVELS // _NPASS
_UNROLL = 4
_NBODY = _NCHUNK // _UNROLL

_P2 = np.int32(np.uint32(2654435761))
_P3 = np.int32(np.uint32(805459861))


def _tec_body(x_hbm, tbl_hbm, out_hbm, tblv, xc0, xc1, st0, st1,
              xsem0, xsem1, osem0, osem1):
    wid = lax.axis_index("s") * np.int32(_NC) + lax.axis_index("c")
    bstart = wid * np.int32(_PPW // 128)
    xcs = (xc0, xc1)
    stages = (st0, st1)
    xsems = (xsem0, xsem1)
    osems = (osem0, osem1)

    for p in range(_NPASS):
        pltpu.sync_copy(tbl_hbm.at[pl.ds(p * _LPP * _HASH, _LPP * _HASH)],
                        tblv)

        def body(bi, carry, p=p):
            cb0 = bstart + bi * np.int32(_CB * _UNROLL)
            cbs = [cb0 + np.int32(u * _CB) for u in range(_UNROLL)]
            cpx = [None] * _UNROLL
            cpo = [None] * _UNROLL
            cpx[0] = pltpu.async_copy(x_hbm.at[pl.ds(cbs[0], _CB), :, :],
                                      xcs[0], xsems[0])
            for u in range(_UNROLL):
                if u + 1 < _UNROLL:
                    cpx[u + 1] = pltpu.async_copy(
                        x_hbm.at[pl.ds(cbs[u + 1], _CB), :, :],
                        xcs[(u + 1) % 2], xsems[(u + 1) % 2])
                cpx[u].wait()
                if u >= 2:
                    cpo[u - 2].wait()

                def group_body(g, carry2, u=u, p=p):
                    xc = xcs[u % 2]
                    stage = stages[u % 2]
                    b = g >> np.int32(3)
                    off = (g & np.int32(7)) * np.int32(16)
                    xr = xc[b, 0, pl.ds(off, 16)]
                    yr = xc[b, 1, pl.ds(off, 16)]
                    zr = xc[b, 2, pl.ds(off, 16)]
                    for j in range(_LPP):
                        r = jnp.float32(_RES[p * _LPP + j])
                        ix = (xr * r).astype(jnp.int32)
                        iy = (yr * r).astype(jnp.int32)
                        iz = (zr * r).astype(jnp.int32)
                        h = ((ix ^ (iy * _P2) ^ (iz * _P3))
                             & jnp.int32(_HASH - 1))
                        packed = plsc.load_gather(
                            tblv, [h + jnp.int32(j * _HASH)])
                        lo = plsc.bitcast(packed << jnp.int32(16),
                                          jnp.float32)
                        hi = plsc.bitcast(packed & jnp.int32(-65536),
                                          jnp.float32)
                        stage[b, 2 * j, pl.ds(off, 16)] = lo
                        stage[b, 2 * j + 1, pl.ds(off, 16)] = hi
                    return carry2

                lax.fori_loop(np.int32(0), np.int32(_G), group_body,
                              np.int32(0))
                cpo[u] = pltpu.async_copy(
                    stages[u % 2],
                    out_hbm.at[p, pl.ds(cbs[u], _CB), :, :],
                    osems[u % 2])
            cpo[_UNROLL - 2].wait()
            cpo[_UNROLL - 1].wait()
            return carry

        lax.fori_loop(np.int32(0), np.int32(_NBODY), body, np.int32(0))


def _pack_tables(tables):
    tb = jax.lax.bitcast_convert_type(tables.astype(jnp.bfloat16), jnp.uint16)
    packed = (tb[..., 0].astype(jnp.uint32)
              | (tb[..., 1].astype(jnp.uint32) << jnp.uint32(16)))
    return jax.lax.bitcast_convert_type(packed, jnp.int32).reshape(
        _NUM_LEVELS * _HASH)


def kernel(x, tables):
    with jax.enable_x64(False):
        return _run(x, tables)


def _run(x, tables):
    tbl = _pack_tables(tables)
    x4 = jnp.pad(x, ((0, 0), (0, 1)))
    xv = x4.T.reshape(4, _NB, 128).transpose((1, 0, 2))
    mesh = plsc.VectorSubcoreMesh(core_axis_name="c", subcore_axis_name="s")
    run = pl.kernel(
        _tec_body,
        out_type=jax.ShapeDtypeStruct((_NPASS, _NB, 8, 128), jnp.float32),
        mesh=mesh,
        scratch_types=[
            pltpu.VMEM((_LPP * _HASH,), jnp.int32),
            pltpu.VMEM((_CB, 4, 128), jnp.float32),
            pltpu.VMEM((_CB, 4, 128), jnp.float32),
            pltpu.VMEM((_CB, 8, 128), jnp.float32),
            pltpu.VMEM((_CB, 8, 128), jnp.float32),
            pltpu.SemaphoreType.DMA,
            pltpu.SemaphoreType.DMA,
            pltpu.SemaphoreType.DMA,
            pltpu.SemaphoreType.DMA,
        ],
        compiler_params=pltpu.CompilerParams(needs_layout_passes=False,
                                             use_tc_tiling_on_sc=False),
    )
    o4 = run(xv, tbl)
    return o4.transpose((1, 3, 0, 2)).reshape(_N, 2 * _NUM_LEVELS)


# speedup vs baseline: 165.0694x; 1.0193x over previous
import numpy as np
import jax
import jax.numpy as jnp
from jax import lax
from jax.experimental import pallas as pl
from jax.experimental.pallas import tpu as pltpu
from jax.experimental.pallas import tpu_sc as plsc

_NUM_LEVELS = 12
_HASH = 2 ** 14
_RES = [int(16 * np.exp((np.log(512) - np.log(16)) / (_NUM_LEVELS - 1)) ** i)
        for i in range(_NUM_LEVELS)]
_N = 1048576
_NB = _N // 128
_NC = 2
_NS = 16
_NW = _NC * _NS
_PPW = _N // _NW
_C = 2048
_CB = _C // 128
_NCHUNK = _PPW // _C
_G = _C // 16
_NPASS = 3
_LPP = _NUM_LEVELS // _NPASS
_UNROLL = 4
_NBODY = _NCHUNK // _UNROLL

_P2 = np.int32(np.uint32(2654435761))
_P3 = np.int32(np.uint32(805459861))


def _tec_body(x_hbm, tbl_hbm, out_hbm, tblv, xc0, xc1, st0, st1,
              xsem0, xsem1, osem0, osem1):
    wid = lax.axis_index("s") * np.int32(_NC) + lax.axis_index("c")
    bstart = wid * np.int32(_PPW // 128)
    xcs = (xc0, xc1)
    stages = (st0, st1)
    xsems = (xsem0, xsem1)
    osems = (osem0, osem1)

    for p in range(_NPASS):
        pltpu.sync_copy(tbl_hbm.at[pl.ds(p * _LPP * _HASH, _LPP * _HASH)],
                        tblv)

        def body(bi, carry, p=p):
            cb0 = bstart + bi * np.int32(_CB * _UNROLL)
            cbs = [cb0 + np.int32(u * _CB) for u in range(_UNROLL)]
            cpx = [None] * _UNROLL
            cpo = [None] * _UNROLL
            cpx[0] = pltpu.async_copy(x_hbm.at[pl.ds(cbs[0], _CB), :, :],
                                      xcs[0], xsems[0])
            for u in range(_UNROLL):
                if u + 1 < _UNROLL:
                    cpx[u + 1] = pltpu.async_copy(
                        x_hbm.at[pl.ds(cbs[u + 1], _CB), :, :],
                        xcs[(u + 1) % 2], xsems[(u + 1) % 2])
                cpx[u].wait()
                if u >= 2:
                    cpo[u - 2].wait()

                def group_body(g, carry2, u=u, p=p):
                    xc = xcs[u % 2]
                    stage = stages[u % 2]
                    b = g >> np.int32(3)
                    off = (g & np.int32(7)) * np.int32(16)
                    xr = xc[b, 0, pl.ds(off, 16)]
                    yr = xc[b, 1, pl.ds(off, 16)]
                    zr = xc[b, 2, pl.ds(off, 16)]
                    for j in range(_LPP):
                        r = jnp.float32(_RES[p * _LPP + j])
                        ix = (xr * r).astype(jnp.int32)
                        iy = (yr * r).astype(jnp.int32)
                        iz = (zr * r).astype(jnp.int32)
                        h = ((ix ^ (iy * _P2) ^ (iz * _P3))
                             & jnp.int32(_HASH - 1))
                        packed = plsc.load_gather(
                            tblv, [h + jnp.int32(j * _HASH)])
                        lo = plsc.bitcast(packed << jnp.int32(16),
                                          jnp.float32)
                        hi = plsc.bitcast(packed & jnp.int32(-65536),
                                          jnp.float32)
                        stage[b, 2 * j, pl.ds(off, 16)] = lo
                        stage[b, 2 * j + 1, pl.ds(off, 16)] = hi
                    return carry2

                lax.fori_loop(np.int32(0), np.int32(_G), group_body,
                              np.int32(0), unroll=2)
                cpo[u] = pltpu.async_copy(
                    stages[u % 2],
                    out_hbm.at[p, pl.ds(cbs[u], _CB), :, :],
                    osems[u % 2])
            cpo[_UNROLL - 2].wait()
            cpo[_UNROLL - 1].wait()
            return carry

        lax.fori_loop(np.int32(0), np.int32(_NBODY), body, np.int32(0))


def _pack_tables(tables):
    tb = jax.lax.bitcast_convert_type(tables.astype(jnp.bfloat16), jnp.uint16)
    packed = (tb[..., 0].astype(jnp.uint32)
              | (tb[..., 1].astype(jnp.uint32) << jnp.uint32(16)))
    return jax.lax.bitcast_convert_type(packed, jnp.int32).reshape(
        _NUM_LEVELS * _HASH)


def kernel(x, tables):
    with jax.enable_x64(False):
        return _run(x, tables)


def _run(x, tables):
    tbl = _pack_tables(tables)
    x4 = jnp.pad(x, ((0, 0), (0, 1)))
    xv = x4.T.reshape(4, _NB, 128).transpose((1, 0, 2))
    mesh = plsc.VectorSubcoreMesh(core_axis_name="c", subcore_axis_name="s")
    run = pl.kernel(
        _tec_body,
        out_type=jax.ShapeDtypeStruct((_NPASS, _NB, 8, 128), jnp.float32),
        mesh=mesh,
        scratch_types=[
            pltpu.VMEM((_LPP * _HASH,), jnp.int32),
            pltpu.VMEM((_CB, 4, 128), jnp.float32),
            pltpu.VMEM((_CB, 4, 128), jnp.float32),
            pltpu.VMEM((_CB, 8, 128), jnp.float32),
            pltpu.VMEM((_CB, 8, 128), jnp.float32),
            pltpu.SemaphoreType.DMA,
            pltpu.SemaphoreType.DMA,
            pltpu.SemaphoreType.DMA,
            pltpu.SemaphoreType.DMA,
        ],
        compiler_params=pltpu.CompilerParams(needs_layout_passes=False,
                                             use_tc_tiling_on_sc=False),
    )
    o4 = run(xv, tbl)
    return o4.transpose((1, 3, 0, 2)).reshape(_N, 2 * _NUM_LEVELS)


# speedup vs baseline: 289.9776x; 1.7567x over previous
import numpy as np
import jax
import jax.numpy as jnp
from jax import lax
from jax.experimental import pallas as pl
from jax.experimental.pallas import tpu as pltpu
from jax.experimental.pallas import tpu_sc as plsc

_NUM_LEVELS = 12
_HASH = 2 ** 14
_RES = [int(16 * np.exp((np.log(512) - np.log(16)) / (_NUM_LEVELS - 1)) ** i)
        for i in range(_NUM_LEVELS)]
_N = 1048576
_NB = _N // 128
_NC = 2
_NS = 16
_NW = _NC * _NS
_PPW = _N // _NW
_C = 2048
_CB = _C // 128
_NCHUNK = _PPW // _C
_G = _C // 16
_NPASS = 3
_LPP = _NUM_LEVELS // _NPASS
_UNROLL = 4
_NBODY = _NCHUNK // _UNROLL

_P2 = np.int32(np.uint32(2654435761))
_P3 = np.int32(np.uint32(805459861))


def _tec_body(x_hbm, tbl_hbm, out_hbm, tblv, xc0, xc1, st0, st1,
              xsem0, xsem1, osem0, osem1):
    wid = lax.axis_index("s") * np.int32(_NC) + lax.axis_index("c")
    bstart = wid * np.int32(_PPW // 128)
    xcs = (xc0, xc1)
    stages = (st0, st1)
    xsems = (xsem0, xsem1)
    osems = (osem0, osem1)

    for p in range(_NPASS):
        pltpu.sync_copy(tbl_hbm.at[pl.ds(p * _LPP * _HASH, _LPP * _HASH)],
                        tblv)

        def body(bi, carry, p=p):
            cb0 = bstart + bi * np.int32(_CB * _UNROLL)
            cbs = [cb0 + np.int32(u * _CB) for u in range(_UNROLL)]
            cpx = [None] * _UNROLL
            cpo = [None] * _UNROLL
            cpx[0] = pltpu.async_copy(x_hbm.at[pl.ds(cbs[0], _CB), :, :],
                                      xcs[0], xsems[0])
            for u in range(_UNROLL):
                if u + 1 < _UNROLL:
                    cpx[u + 1] = pltpu.async_copy(
                        x_hbm.at[pl.ds(cbs[u + 1], _CB), :, :],
                        xcs[(u + 1) % 2], xsems[(u + 1) % 2])
                cpx[u].wait()
                if u >= 2:
                    cpo[u - 2].wait()

                @plsc.parallel_loop(0, _G, step=1, unroll=4)
                def group_body(g, u=u, p=p):
                    xc = xcs[u % 2]
                    stage = stages[u % 2]
                    b = g >> np.int32(3)
                    off = (g & np.int32(7)) * np.int32(16)
                    xr = xc[b, 0, pl.ds(off, 16)]
                    yr = xc[b, 1, pl.ds(off, 16)]
                    zr = xc[b, 2, pl.ds(off, 16)]
                    for j in range(_LPP):
                        r = jnp.float32(_RES[p * _LPP + j])
                        ix = (xr * r).astype(jnp.int32)
                        iy = (yr * r).astype(jnp.int32)
                        iz = (zr * r).astype(jnp.int32)
                        h = ((ix ^ (iy * _P2) ^ (iz * _P3))
                             & jnp.int32(_HASH - 1))
                        packed = plsc.load_gather(
                            tblv, [h + jnp.int32(j * _HASH)])
                        lo = plsc.bitcast(packed << jnp.int32(16),
                                          jnp.float32)
                        hi = plsc.bitcast(packed & jnp.int32(-65536),
                                          jnp.float32)
                        stage[b, 2 * j, pl.ds(off, 16)] = lo
                        stage[b, 2 * j + 1, pl.ds(off, 16)] = hi

                cpo[u] = pltpu.async_copy(
                    stages[u % 2],
                    out_hbm.at[p, pl.ds(cbs[u], _CB), :, :],
                    osems[u % 2])
            cpo[_UNROLL - 2].wait()
            cpo[_UNROLL - 1].wait()
            return carry

        lax.fori_loop(np.int32(0), np.int32(_NBODY), body, np.int32(0))


def _pack_tables(tables):
    tb = jax.lax.bitcast_convert_type(tables.astype(jnp.bfloat16), jnp.uint16)
    packed = (tb[..., 0].astype(jnp.uint32)
              | (tb[..., 1].astype(jnp.uint32) << jnp.uint32(16)))
    return jax.lax.bitcast_convert_type(packed, jnp.int32).reshape(
        _NUM_LEVELS * _HASH)


def kernel(x, tables):
    with jax.enable_x64(False):
        return _run(x, tables)


def _run(x, tables):
    tbl = _pack_tables(tables)
    x4 = jnp.pad(x, ((0, 0), (0, 1)))
    xv = x4.T.reshape(4, _NB, 128).transpose((1, 0, 2))
    mesh = plsc.VectorSubcoreMesh(core_axis_name="c", subcore_axis_name="s")
    run = pl.kernel(
        _tec_body,
        out_type=jax.ShapeDtypeStruct((_NPASS, _NB, 8, 128), jnp.float32),
        mesh=mesh,
        scratch_types=[
            pltpu.VMEM((_LPP * _HASH,), jnp.int32),
            pltpu.VMEM((_CB, 4, 128), jnp.float32),
            pltpu.VMEM((_CB, 4, 128), jnp.float32),
            pltpu.VMEM((_CB, 8, 128), jnp.float32),
            pltpu.VMEM((_CB, 8, 128), jnp.float32),
            pltpu.SemaphoreType.DMA,
            pltpu.SemaphoreType.DMA,
            pltpu.SemaphoreType.DMA,
            pltpu.SemaphoreType.DMA,
        ],
        compiler_params=pltpu.CompilerParams(needs_layout_passes=False,
                                             use_tc_tiling_on_sc=False),
    )
    o4 = run(xv, tbl)
    return o4.transpose((1, 3, 0, 2)).reshape(_N, 2 * _NUM_LEVELS)
